# baseline (speedup 1.0000x reference)
import jax
import jax.numpy as jnp
from jax.experimental import pallas as pl
from jax.experimental.pallas import tpu as pltpu

H = 128
CHUNK = 1000


def _gru_body(x_hbm, wih_ref, whh_ref, bih_ref, bhh_ref, o_hbm,
              w_ref, b_ref, ones_ref, xbuf, obuf, in_sem, out_sem):
    n_chunks = x_hbm.shape[0] // CHUNK

    wih = wih_ref[...]
    whh = whh_ref[...]
    w_rz = 0.5 * (wih[0:2 * H] + whh[0:2 * H])
    w_ref[:, 0:2 * H] = w_rz.T
    w_ref[:, 2 * H:3 * H] = wih[2 * H:3 * H].T
    w_ref[:, 3 * H:4 * H] = whh[2 * H:3 * H].T
    b_ref[:, 0:2 * H] = 0.5 * (bih_ref[:, 0:2 * H] + bhh_ref[:, 0:2 * H])
    b_ref[:, 2 * H:3 * H] = bih_ref[:, 2 * H:3 * H]
    b_ref[:, 3 * H:4 * H] = bhh_ref[:, 2 * H:3 * H]
    ones_ref[...] = jnp.ones((H, H), jnp.float32)

    def copy_in(k):
        return pltpu.make_async_copy(
            x_hbm.at[pl.ds(k * CHUNK, CHUNK), :], xbuf.at[k % 2], in_sem.at[k % 2])

    def copy_out(k):
        return pltpu.make_async_copy(
            obuf.at[k % 2], o_hbm.at[pl.ds(k * CHUNK, CHUNK), :], out_sem.at[k % 2])

    copy_in(0).start()

    for k in range(n_chunks):
        if k + 1 < n_chunks:
            copy_in(k + 1).start()
        copy_in(k).wait()
        if k >= 2:
            copy_out(k - 2).wait()
        x = xbuf[k % 2]
        s = jnp.dot(x * x, ones_ref[...], preferred_element_type=jnp.float32)
        h = x * jax.lax.rsqrt(jnp.maximum(s, 1e-24))
        g = jnp.dot(h, w_ref[...], preferred_element_type=jnp.float32) + b_ref[...]
        r = 0.5 * (1.0 + jnp.tanh(g[:, 0:H]))
        z = 0.5 * (1.0 + jnp.tanh(g[:, H:2 * H]))
        c = jnp.tanh(g[:, 2 * H:3 * H] + r * g[:, 3 * H:4 * H])
        obuf[k % 2] = c + z * (h - c)
        copy_out(k).start()

    if n_chunks >= 2:
        copy_out(n_chunks - 2).wait()
    copy_out(n_chunks - 1).wait()


def kernel(dynamic_emb, emb_rel, W_ih_rel, W_hh_rel, b_ih_rel, b_hh_rel,
           W_ih_ent, W_hh_ent, b_ih_ent, b_hh_ent, r_to_e, seg_ids):
    N, Hd = dynamic_emb.shape
    out = pl.pallas_call(
        _gru_body,
        grid=(1,),
        in_specs=[
            pl.BlockSpec(memory_space=pl.ANY),
            pl.BlockSpec((3 * H, Hd), lambda i: (0, 0)),
            pl.BlockSpec((3 * H, Hd), lambda i: (0, 0)),
            pl.BlockSpec((1, 3 * H), lambda i: (0, 0)),
            pl.BlockSpec((1, 3 * H), lambda i: (0, 0)),
        ],
        out_specs=pl.BlockSpec(memory_space=pl.ANY),
        out_shape=jax.ShapeDtypeStruct((N, Hd), jnp.float32),
        scratch_shapes=[
            pltpu.VMEM((Hd, 4 * H), jnp.float32),
            pltpu.VMEM((1, 4 * H), jnp.float32),
            pltpu.VMEM((H, H), jnp.float32),
            pltpu.VMEM((2, CHUNK, Hd), jnp.float32),
            pltpu.VMEM((2, CHUNK, Hd), jnp.float32),
            pltpu.SemaphoreType.DMA((2,)),
            pltpu.SemaphoreType.DMA((2,)),
        ],
    )(dynamic_emb, W_ih_ent, W_hh_ent, b_ih_ent[None, :], b_hh_ent[None, :])
    return out


# speedup vs baseline: 1.1412x; 1.1412x over previous
import jax
import jax.numpy as jnp
from jax.experimental import pallas as pl
from jax.experimental.pallas import tpu as pltpu

H = 128


def _gru_body(x_ref, wih_ref, whh_ref, bih_ref, bhh_ref, o_ref,
              w_ref, b_ref, ones_ref):
    i = pl.program_id(0)

    @pl.when(i == 0)
    def _init():
        wih = wih_ref[...]
        whh = whh_ref[...]
        w_rz = 0.5 * (wih[0:2 * H] + whh[0:2 * H])
        w_ref[:, 0:2 * H] = w_rz.T.astype(jnp.bfloat16)
        w_ref[:, 2 * H:3 * H] = wih[2 * H:3 * H].T.astype(jnp.bfloat16)
        w_ref[:, 3 * H:4 * H] = whh[2 * H:3 * H].T.astype(jnp.bfloat16)
        b_ref[:, 0:2 * H] = 0.5 * (bih_ref[:, 0:2 * H] + bhh_ref[:, 0:2 * H])
        b_ref[:, 2 * H:3 * H] = bih_ref[:, 2 * H:3 * H]
        b_ref[:, 3 * H:4 * H] = bhh_ref[:, 2 * H:3 * H]
        ones_ref[...] = jnp.ones((H, H), jnp.bfloat16)

    x = x_ref[...]
    s = jnp.dot((x * x).astype(jnp.bfloat16), ones_ref[...],
                preferred_element_type=jnp.float32)
    h = x * jax.lax.rsqrt(jnp.maximum(s, 1e-24))
    g = jnp.dot(h.astype(jnp.bfloat16), w_ref[...],
                preferred_element_type=jnp.float32) + b_ref[...]
    r = 0.5 * (1.0 + jnp.tanh(g[:, 0:H]))
    z = 0.5 * (1.0 + jnp.tanh(g[:, H:2 * H]))
    c = jnp.tanh(g[:, 2 * H:3 * H] + r * g[:, 3 * H:4 * H])
    o_ref[...] = c + z * (h - c)


def kernel(dynamic_emb, emb_rel, W_ih_rel, W_hh_rel, b_ih_rel, b_hh_rel,
           W_ih_ent, W_hh_ent, b_ih_ent, b_hh_ent, r_to_e, seg_ids):
    N, Hd = dynamic_emb.shape
    B = 2000
    out = pl.pallas_call(
        _gru_body,
        grid=(N // B,),
        in_specs=[
            pl.BlockSpec((B, Hd), lambda i: (i, 0)),
            pl.BlockSpec((3 * H, Hd), lambda i: (0, 0)),
            pl.BlockSpec((3 * H, Hd), lambda i: (0, 0)),
            pl.BlockSpec((1, 3 * H), lambda i: (0, 0)),
            pl.BlockSpec((1, 3 * H), lambda i: (0, 0)),
        ],
        out_specs=pl.BlockSpec((B, Hd), lambda i: (i, 0)),
        out_shape=jax.ShapeDtypeStruct((N, Hd), jnp.float32),
        scratch_shapes=[
            pltpu.VMEM((Hd, 4 * H), jnp.bfloat16),
            pltpu.VMEM((1, 4 * H), jnp.float32),
            pltpu.VMEM((H, H), jnp.bfloat16),
        ],
        compiler_params=pltpu.CompilerParams(
            dimension_semantics=("parallel",)),
    )(dynamic_emb, W_ih_ent, W_hh_ent, b_ih_ent[None, :], b_hh_ent[None, :])
    return out


# speedup vs baseline: 1.1589x; 1.0155x over previous
import jax
import jax.numpy as jnp
from jax.experimental import pallas as pl
from jax.experimental.pallas import tpu as pltpu

H = 128


def _gru_body(x_ref, wih_ref, whh_ref, bih_ref, bhh_ref, o_ref,
              w_ref, b_ref, ones_ref):
    i = pl.program_id(0)

    @pl.when(i == 0)
    def _init():
        wih = wih_ref[...]
        whh = whh_ref[...]
        w_rz = 0.5 * (wih[0:2 * H] + whh[0:2 * H])
        w_ref[:, 0:2 * H] = w_rz.T.astype(jnp.bfloat16)
        w_ref[:, 2 * H:3 * H] = wih[2 * H:3 * H].T.astype(jnp.bfloat16)
        w_ref[:, 3 * H:4 * H] = whh[2 * H:3 * H].T.astype(jnp.bfloat16)
        b_ref[:, 0:2 * H] = (0.5 * (bih_ref[:, 0:2 * H]
                                    + bhh_ref[:, 0:2 * H])).astype(jnp.bfloat16)
        b_ref[:, 2 * H:3 * H] = bih_ref[:, 2 * H:3 * H].astype(jnp.bfloat16)
        b_ref[:, 3 * H:4 * H] = bhh_ref[:, 2 * H:3 * H].astype(jnp.bfloat16)
        ones_ref[...] = jnp.ones((H, H), jnp.bfloat16)

    x = x_ref[...]
    xb = x.astype(jnp.bfloat16)
    s = jnp.dot(xb * xb, ones_ref[...], preferred_element_type=jnp.float32)
    h = (x * jax.lax.rsqrt(jnp.maximum(s, 1e-24))).astype(jnp.bfloat16)
    g = jnp.dot(h, w_ref[...],
                preferred_element_type=jnp.float32).astype(jnp.bfloat16) + b_ref[...]
    one = jnp.bfloat16(1.0)
    half = jnp.bfloat16(0.5)
    r = half * (one + jnp.tanh(g[:, 0:H]))
    z = half * (one + jnp.tanh(g[:, H:2 * H]))
    c = jnp.tanh(g[:, 2 * H:3 * H] + r * g[:, 3 * H:4 * H])
    o_ref[...] = (c + z * (h - c)).astype(jnp.float32)


def kernel(dynamic_emb, emb_rel, W_ih_rel, W_hh_rel, b_ih_rel, b_hh_rel,
           W_ih_ent, W_hh_ent, b_ih_ent, b_hh_ent, r_to_e, seg_ids):
    N, Hd = dynamic_emb.shape
    B = 2000
    out = pl.pallas_call(
        _gru_body,
        grid=(N // B,),
        in_specs=[
            pl.BlockSpec((B, Hd), lambda i: (i, 0)),
            pl.BlockSpec((3 * H, Hd), lambda i: (0, 0)),
            pl.BlockSpec((3 * H, Hd), lambda i: (0, 0)),
            pl.BlockSpec((1, 3 * H), lambda i: (0, 0)),
            pl.BlockSpec((1, 3 * H), lambda i: (0, 0)),
        ],
        out_specs=pl.BlockSpec((B, Hd), lambda i: (i, 0)),
        out_shape=jax.ShapeDtypeStruct((N, Hd), jnp.float32),
        scratch_shapes=[
            pltpu.VMEM((Hd, 4 * H), jnp.bfloat16),
            pltpu.VMEM((1, 4 * H), jnp.bfloat16),
            pltpu.VMEM((H, H), jnp.bfloat16),
        ],
        compiler_params=pltpu.CompilerParams(
            dimension_semantics=("parallel",)),
    )(dynamic_emb, W_ih_ent, W_hh_ent, b_ih_ent[None, :], b_hh_ent[None, :])
    return out


# speedup vs baseline: 1.5076x; 1.3009x over previous
import jax
import jax.numpy as jnp
from jax.experimental import pallas as pl
from jax.experimental.pallas import tpu as pltpu

H = 128


def _gru_body(x_ref, wih_ref, whh_ref, bih_ref, bhh_ref, o_ref,
              w_ref, b_ref, ones_ref):
    i = pl.program_id(0)

    @pl.when(i == 0)
    def _init():
        wih = wih_ref[...]
        whh = whh_ref[...]
        w_rz = 0.5 * (wih[0:2 * H] + whh[0:2 * H])
        w_ref[:, 0:2 * H] = w_rz.T
        w_ref[:, 2 * H:3 * H] = wih[2 * H:3 * H].T
        w_ref[:, 3 * H:4 * H] = whh[2 * H:3 * H].T
        bih = bih_ref[...][None, :]
        bhh = bhh_ref[...][None, :]
        b_ref[:, 0:2 * H] = 0.5 * (bih[:, 0:2 * H] + bhh[:, 0:2 * H])
        b_ref[:, 2 * H:3 * H] = bih[:, 2 * H:3 * H]
        b_ref[:, 3 * H:4 * H] = bhh[:, 2 * H:3 * H]
        ones_ref[...] = jnp.ones((H, H), jnp.float32)

    x = x_ref[...]
    s = jnp.dot(x * x, ones_ref[...], preferred_element_type=jnp.float32)
    h = x * jax.lax.rsqrt(jnp.maximum(s, 1e-24))
    g = jnp.dot(h, w_ref[...], preferred_element_type=jnp.float32) + b_ref[...]
    r = 0.5 * (1.0 + jnp.tanh(g[:, 0:H]))
    z = 0.5 * (1.0 + jnp.tanh(g[:, H:2 * H]))
    c = jnp.tanh(g[:, 2 * H:3 * H] + r * g[:, 3 * H:4 * H])
    o_ref[...] = c + z * (h - c)


def kernel(dynamic_emb, emb_rel, W_ih_rel, W_hh_rel, b_ih_rel, b_hh_rel,
           W_ih_ent, W_hh_ent, b_ih_ent, b_hh_ent, r_to_e, seg_ids):
    N, Hd = dynamic_emb.shape
    B = 2000
    out = pl.pallas_call(
        _gru_body,
        grid=(N // B,),
        in_specs=[
            pl.BlockSpec((B, Hd), lambda i: (i, 0)),
            pl.BlockSpec((3 * H, Hd), lambda i: (0, 0)),
            pl.BlockSpec((3 * H, Hd), lambda i: (0, 0)),
            pl.BlockSpec((3 * H,), lambda i: (0,)),
            pl.BlockSpec((3 * H,), lambda i: (0,)),
        ],
        out_specs=pl.BlockSpec((B, Hd), lambda i: (i, 0)),
        out_shape=jax.ShapeDtypeStruct((N, Hd), jnp.float32),
        scratch_shapes=[
            pltpu.VMEM((Hd, 4 * H), jnp.float32),
            pltpu.VMEM((1, 4 * H), jnp.float32),
            pltpu.VMEM((H, H), jnp.float32),
        ],
        compiler_params=pltpu.CompilerParams(
            dimension_semantics=("parallel",)),
    )(dynamic_emb, W_ih_ent, W_hh_ent, b_ih_ent, b_hh_ent)
    return out
